# baseline (speedup 1.0000x reference)
import jax
import jax.numpy as jnp
from jax.experimental import pallas as pl
from jax.experimental.pallas import tpu as pltpu

B, T, D, H = 16, 512, 256, 256
G = 4 * H
C = 64


def _lstm_kernel(x_ref, slen_ref, wih_ref, whh_ref, b_ref,
                 out_ref, hs_ref, cs_ref, xp_ref):
    lens = slen_ref[...]

    out_ref[...] = jnp.zeros_like(out_ref)

    maxlen = jnp.max(lens)
    nchunks = (maxlen + (C - 1)) // C

    bias = b_ref[...]
    wih = wih_ref[...].astype(jnp.bfloat16)
    whh = whh_ref[...].astype(jnp.bfloat16)

    def chunk_body(ci, carry):
        t0 = ci * C
        xc = x_ref[pl.ds(t0, C), :, :]
        xp_ref[...] = (jnp.dot(xc.reshape(C * B, D).astype(jnp.bfloat16), wih,
                               preferred_element_type=jnp.float32)
                       + bias).astype(jnp.bfloat16)

        def step(cc, carry2):
            h2, c2 = carry2
            t = t0 + cc
            gates = xp_ref[pl.ds(cc * B, B), :].astype(jnp.float32) + jnp.dot(
                h2.astype(jnp.bfloat16), whh, preferred_element_type=jnp.float32)
            sig = jax.nn.sigmoid(gates[:, :3 * H])
            i_g = sig[:, :H]
            f_g = sig[:, H:2 * H]
            o_g = sig[:, 2 * H:]
            g_g = jnp.tanh(gates[:, 3 * H:])
            nc = f_g * c2 + i_g * g_g
            nh = o_g * jnp.tanh(nc)
            active = t < lens
            c3 = jnp.where(active, nc, c2)
            h3 = jnp.where(active, nh, h2)
            outv = jnp.where(active, nh, 0.0)
            out_ref[pl.ds(t, 1), :, :] = outv[None]
            return (h3, c3)

        return jax.lax.fori_loop(0, C, step, carry, unroll=8)

    z = jnp.zeros((B, H), jnp.float32)
    hf, cf = jax.lax.fori_loop(0, nchunks, chunk_body, (z, z))
    hs_ref[...] = hf
    cs_ref[...] = cf


def _reorder_gates(w):
    return jnp.concatenate([w[..., :2 * H], w[..., 3 * H:], w[..., 2 * H:3 * H]],
                           axis=-1)


@jax.jit
def kernel(inputs, mask, W_ih, W_hh, b_ih, b_hh):
    mask = mask.astype(jnp.int32)
    lengths = mask.sum(-1)
    permutation = jnp.argsort(-lengths)
    restoration = jnp.argsort(permutation).astype(jnp.int32)
    sorted_lengths = lengths[permutation]

    x_tm = jnp.transpose(inputs, (1, 0, 2))[:, permutation, :]
    out_tm, hs, cs = pl.pallas_call(
        _lstm_kernel,
        out_shape=[
            jax.ShapeDtypeStruct((T, B, H), jnp.float32),
            jax.ShapeDtypeStruct((B, H), jnp.float32),
            jax.ShapeDtypeStruct((B, H), jnp.float32),
        ],
        scratch_shapes=[pltpu.VMEM((C * B, G), jnp.bfloat16)],
    )(x_tm, sorted_lengths[:, None], _reorder_gates(W_ih.T),
      _reorder_gates(W_hh.T), _reorder_gates((b_ih + b_hh)[None, :]))

    outputs = jnp.transpose(out_tm, (1, 0, 2))
    return outputs, hs[None], cs[None], restoration


# speedup vs baseline: 1.0028x; 1.0028x over previous
import jax
import jax.numpy as jnp
from jax.experimental import pallas as pl
from jax.experimental.pallas import tpu as pltpu

B, T, D, H = 16, 512, 256, 256
G = 4 * H
C = 64


def _lstm_kernel(x_ref, slen_ref, wih_ref, whh_ref, b_ref,
                 out_ref, hs_ref, cs_ref, xp_ref):
    lens = slen_ref[...]

    out_ref[...] = jnp.zeros_like(out_ref)

    maxlen = jnp.max(lens)
    nchunks = (maxlen + (C - 1)) // C

    bias = b_ref[...]
    wih = wih_ref[...].astype(jnp.bfloat16)
    whh = whh_ref[...].astype(jnp.bfloat16)

    def chunk_body(ci, carry):
        t0 = ci * C
        xc = x_ref[pl.ds(t0, C), :, :]
        xp_ref[...] = (jnp.dot(xc.reshape(C * B, D).astype(jnp.bfloat16), wih,
                               preferred_element_type=jnp.float32)
                       + bias).astype(jnp.bfloat16)

        def step(cc, carry2):
            h2, c2 = carry2
            t = t0 + cc
            gates = xp_ref[pl.ds(cc * B, B), :].astype(jnp.float32) + jnp.dot(
                h2.astype(jnp.bfloat16), whh, preferred_element_type=jnp.float32)
            sig = jax.nn.sigmoid(gates[:, :3 * H])
            i_g = sig[:, :H]
            f_g = sig[:, H:2 * H]
            o_g = sig[:, 2 * H:]
            g_g = jnp.tanh(gates[:, 3 * H:])
            nc = f_g * c2 + i_g * g_g
            nh = o_g * jnp.tanh(nc)
            active = t < lens
            c3 = jnp.where(active, nc, c2)
            h3 = jnp.where(active, nh, h2)
            outv = jnp.where(active, nh, 0.0)
            out_ref[pl.ds(t, 1), :, :] = outv[None]
            return (h3, c3)

        return jax.lax.fori_loop(0, C, step, carry, unroll=16)

    z = jnp.zeros((B, H), jnp.float32)
    hf, cf = jax.lax.fori_loop(0, nchunks, chunk_body, (z, z))
    hs_ref[...] = hf
    cs_ref[...] = cf


def _reorder_gates(w):
    return jnp.concatenate([w[..., :2 * H], w[..., 3 * H:], w[..., 2 * H:3 * H]],
                           axis=-1)


@jax.jit
def kernel(inputs, mask, W_ih, W_hh, b_ih, b_hh):
    mask = mask.astype(jnp.int32)
    lengths = mask.sum(-1)
    permutation = jnp.argsort(-lengths)
    restoration = jnp.argsort(permutation).astype(jnp.int32)
    sorted_lengths = lengths[permutation]

    x_tm = jnp.transpose(inputs, (1, 0, 2))[:, permutation, :]
    out_tm, hs, cs = pl.pallas_call(
        _lstm_kernel,
        out_shape=[
            jax.ShapeDtypeStruct((T, B, H), jnp.float32),
            jax.ShapeDtypeStruct((B, H), jnp.float32),
            jax.ShapeDtypeStruct((B, H), jnp.float32),
        ],
        scratch_shapes=[pltpu.VMEM((C * B, G), jnp.bfloat16)],
    )(x_tm, sorted_lengths[:, None], _reorder_gates(W_ih.T),
      _reorder_gates(W_hh.T), _reorder_gates((b_ih + b_hh)[None, :]))

    outputs = jnp.transpose(out_tm, (1, 0, 2))
    return outputs, hs[None], cs[None], restoration


# speedup vs baseline: 1.4781x; 1.4739x over previous
import jax
import jax.numpy as jnp
from jax.experimental import pallas as pl
from jax.experimental.pallas import tpu as pltpu

B, T, D, H = 16, 512, 256, 256
G = 4 * H
C = 64
NCH = T // C


def _lstm_kernel(x_ref, perm_ref, slen_ref, wih_ref, whh_ref, b_ref,
                 out_ref, hs_ref, cs_ref,
                 xb_ref, ob_ref, xp_ref, zb_ref, h_ref, c_ref,
                 in_sems, out_sems, zsem):
    lens = slen_ref[...]
    maxlen = jnp.max(lens)
    nchunks = (maxlen + (C - 1)) // C

    bias = b_ref[...]
    wih = wih_ref[...].astype(jnp.bfloat16)
    whh = whh_ref[...].astype(jnp.bfloat16)

    def in_copy(ci, s, b):
        return pltpu.make_async_copy(
            x_ref.at[perm_ref[b], pl.ds(ci * C, C), :],
            xb_ref.at[s, :, b, :], in_sems.at[s])

    def out_copy(ci, s, b):
        return pltpu.make_async_copy(
            ob_ref.at[s, :, b, :],
            out_ref.at[b, pl.ds(ci * C, C), :], out_sems.at[s])

    @pl.when(nchunks > 0)
    def _prefetch0():
        for b in range(B):
            in_copy(0, 0, b).start()

    zb_ref[...] = jnp.zeros_like(zb_ref)

    def _ztail(ci, carry):
        for b in range(B):
            pltpu.make_async_copy(
                zb_ref, out_ref.at[b, pl.ds(ci * C, C), :], zsem).start()
        return carry

    jax.lax.fori_loop(nchunks, NCH, _ztail, 0)

    h_ref[...] = jnp.zeros_like(h_ref)
    c_ref[...] = jnp.zeros_like(c_ref)

    for ci in range(NCH):
        s = ci % 2

        @pl.when(ci < nchunks)
        def _chunk(ci=ci, s=s):
            if ci + 1 < NCH:
                @pl.when(ci + 1 < nchunks)
                def _prefetch():
                    for b in range(B):
                        in_copy(ci + 1, (ci + 1) % 2, b).start()
            for b in range(B):
                in_copy(ci, s, b).wait()

            xp_ref[...] = (jnp.dot(
                xb_ref[s].reshape(C * B, D).astype(jnp.bfloat16), wih,
                preferred_element_type=jnp.float32) + bias).astype(jnp.bfloat16)

            if ci >= 2:
                for b in range(B):
                    out_copy(ci - 2, s, b).wait()

            t0 = ci * C

            def step(cc, carry2):
                h2, c2 = carry2
                t = t0 + cc
                gates = xp_ref[pl.ds(cc * B, B), :].astype(jnp.float32) + jnp.dot(
                    h2.astype(jnp.bfloat16), whh,
                    preferred_element_type=jnp.float32)
                sig = jax.nn.sigmoid(gates[:, :3 * H])
                i_g = sig[:, :H]
                f_g = sig[:, H:2 * H]
                o_g = sig[:, 2 * H:]
                g_g = jnp.tanh(gates[:, 3 * H:])
                nc = f_g * c2 + i_g * g_g
                nh = o_g * jnp.tanh(nc)
                active = t < lens
                c3 = jnp.where(active, nc, c2)
                h3 = jnp.where(active, nh, h2)
                outv = jnp.where(active, nh, 0.0)
                ob_ref[s, pl.ds(cc, 1), :, :] = outv[None]
                return (h3, c3)

            hf, cf = jax.lax.fori_loop(0, C, step, (h_ref[...], c_ref[...]),
                                       unroll=16)
            h_ref[...] = hf
            c_ref[...] = cf

            for b in range(B):
                out_copy(ci, s, b).start()

    @pl.when(nchunks >= 2)
    def _drain2():
        for b in range(B):
            out_copy(0, 0, b).wait()
            out_copy(0, 1, b).wait()

    @pl.when(nchunks == 1)
    def _drain1():
        for b in range(B):
            out_copy(0, 0, b).wait()

    def _zwait(ci, carry):
        for b in range(B):
            pltpu.make_async_copy(
                zb_ref, out_ref.at[b, pl.ds(0, C), :], zsem).wait()
        return carry

    jax.lax.fori_loop(nchunks, NCH, _zwait, 0)

    hs_ref[...] = h_ref[...]
    cs_ref[...] = c_ref[...]


def _reorder_gates(w):
    return jnp.concatenate([w[..., :2 * H], w[..., 3 * H:], w[..., 2 * H:3 * H]],
                           axis=-1)


@jax.jit
def kernel(inputs, mask, W_ih, W_hh, b_ih, b_hh):
    mask = mask.astype(jnp.int32)
    lengths = mask.sum(-1)
    permutation = jnp.argsort(-lengths)
    restoration = jnp.argsort(permutation).astype(jnp.int32)
    sorted_lengths = lengths[permutation]

    outputs, hs, cs = pl.pallas_call(
        _lstm_kernel,
        in_specs=[
            pl.BlockSpec(memory_space=pltpu.MemorySpace.HBM),
            pl.BlockSpec(memory_space=pltpu.MemorySpace.SMEM),
            pl.BlockSpec(memory_space=pltpu.MemorySpace.VMEM),
            pl.BlockSpec(memory_space=pltpu.MemorySpace.VMEM),
            pl.BlockSpec(memory_space=pltpu.MemorySpace.VMEM),
            pl.BlockSpec(memory_space=pltpu.MemorySpace.VMEM),
        ],
        out_specs=[
            pl.BlockSpec(memory_space=pltpu.MemorySpace.HBM),
            pl.BlockSpec(memory_space=pltpu.MemorySpace.VMEM),
            pl.BlockSpec(memory_space=pltpu.MemorySpace.VMEM),
        ],
        out_shape=[
            jax.ShapeDtypeStruct((B, T, H), jnp.float32),
            jax.ShapeDtypeStruct((B, H), jnp.float32),
            jax.ShapeDtypeStruct((B, H), jnp.float32),
        ],
        scratch_shapes=[
            pltpu.VMEM((2, C, B, D), jnp.float32),
            pltpu.VMEM((2, C, B, H), jnp.float32),
            pltpu.VMEM((C * B, G), jnp.bfloat16),
            pltpu.VMEM((C, H), jnp.float32),
            pltpu.VMEM((B, H), jnp.float32),
            pltpu.VMEM((B, H), jnp.float32),
            pltpu.SemaphoreType.DMA((2,)),
            pltpu.SemaphoreType.DMA((2,)),
            pltpu.SemaphoreType.DMA,
        ],
    )(inputs, permutation.astype(jnp.int32), sorted_lengths[:, None],
      _reorder_gates(W_ih.T), _reorder_gates(W_hh.T),
      _reorder_gates((b_ih + b_hh)[None, :]))

    return outputs, hs[None], cs[None], restoration


# speedup vs baseline: 1.4860x; 1.0054x over previous
import jax
import jax.numpy as jnp
from jax.experimental import pallas as pl
from jax.experimental.pallas import tpu as pltpu

B, T, D, H = 16, 512, 256, 256
G = 4 * H
C = 64
NCH = T // C


def _lstm_kernel(x_ref, perm_ref, slen_ref, wih_ref, whh_ref, bih_ref, bhh_ref,
                 out_ref, hs_ref, cs_ref,
                 xb_ref, ob_ref, xp_ref, zb_ref, h_ref, c_ref,
                 in_sems, out_sems, zsem):
    lens = slen_ref[...]
    maxlen = jnp.max(lens)
    nchunks = (maxlen + (C - 1)) // C

    bias = bih_ref[...] + bhh_ref[...]
    wih = wih_ref[...].astype(jnp.bfloat16)
    whh = whh_ref[...].astype(jnp.bfloat16)

    def in_copy(ci, s, b):
        return pltpu.make_async_copy(
            x_ref.at[perm_ref[b], pl.ds(ci * C, C), :],
            xb_ref.at[s, :, b, :], in_sems.at[s])

    def out_copy(ci, s, b):
        return pltpu.make_async_copy(
            ob_ref.at[s, :, b, :],
            out_ref.at[b, pl.ds(ci * C, C), :], out_sems.at[s])

    @pl.when(nchunks > 0)
    def _prefetch0():
        for b in range(B):
            in_copy(0, 0, b).start()

    zb_ref[...] = jnp.zeros_like(zb_ref)

    def _ztail(ci, carry):
        for b in range(B):
            pltpu.make_async_copy(
                zb_ref, out_ref.at[b, pl.ds(ci * C, C), :], zsem).start()
        return carry

    jax.lax.fori_loop(nchunks, NCH, _ztail, 0)

    h_ref[...] = jnp.zeros_like(h_ref)
    c_ref[...] = jnp.zeros_like(c_ref)

    for ci in range(NCH):
        s = ci % 2

        @pl.when(ci < nchunks)
        def _chunk(ci=ci, s=s):
            if ci + 1 < NCH:
                @pl.when(ci + 1 < nchunks)
                def _prefetch():
                    for b in range(B):
                        in_copy(ci + 1, (ci + 1) % 2, b).start()
            for b in range(B):
                in_copy(ci, s, b).wait()

            xp_ref[...] = (jnp.dot(
                xb_ref[s].reshape(C * B, D).astype(jnp.bfloat16), wih,
                preferred_element_type=jnp.float32) + bias).astype(jnp.bfloat16)

            if ci >= 2:
                for b in range(B):
                    out_copy(ci - 2, s, b).wait()

            t0 = ci * C

            def step(cc, carry2):
                h2, c2 = carry2
                t = t0 + cc
                gates = xp_ref[pl.ds(cc * B, B), :].astype(jnp.float32) + jnp.dot(
                    h2.astype(jnp.bfloat16), whh,
                    preferred_element_type=jnp.float32)
                sig_if = jax.nn.sigmoid(gates[:, :2 * H])
                i_g = sig_if[:, :H]
                f_g = sig_if[:, H:]
                g_g = jnp.tanh(gates[:, 2 * H:3 * H])
                o_g = jax.nn.sigmoid(gates[:, 3 * H:])
                nc = f_g * c2 + i_g * g_g
                nh = o_g * jnp.tanh(nc)
                active = t < lens
                c3 = jnp.where(active, nc, c2)
                h3 = jnp.where(active, nh, h2)
                outv = jnp.where(active, nh, 0.0)
                ob_ref[s, pl.ds(cc, 1), :, :] = outv[None]
                return (h3, c3)

            hf, cf = jax.lax.fori_loop(0, C, step, (h_ref[...], c_ref[...]),
                                       unroll=16)
            h_ref[...] = hf
            c_ref[...] = cf

            for b in range(B):
                out_copy(ci, s, b).start()

    @pl.when(nchunks >= 2)
    def _drain2():
        for b in range(B):
            out_copy(0, 0, b).wait()
            out_copy(0, 1, b).wait()

    @pl.when(nchunks == 1)
    def _drain1():
        for b in range(B):
            out_copy(0, 0, b).wait()

    def _zwait(ci, carry):
        for b in range(B):
            pltpu.make_async_copy(
                zb_ref, out_ref.at[b, pl.ds(0, C), :], zsem).wait()
        return carry

    jax.lax.fori_loop(nchunks, NCH, _zwait, 0)

    hs_ref[...] = h_ref[...]
    cs_ref[...] = c_ref[...]


@jax.jit
def kernel(inputs, mask, W_ih, W_hh, b_ih, b_hh):
    mask = mask.astype(jnp.int32)
    lengths = mask.sum(-1)
    permutation = jnp.argsort(-lengths)
    restoration = jnp.argsort(permutation).astype(jnp.int32)
    sorted_lengths = lengths[permutation]

    outputs, hs, cs = pl.pallas_call(
        _lstm_kernel,
        in_specs=[
            pl.BlockSpec(memory_space=pltpu.MemorySpace.HBM),
            pl.BlockSpec(memory_space=pltpu.MemorySpace.SMEM),
            pl.BlockSpec(memory_space=pltpu.MemorySpace.VMEM),
            pl.BlockSpec(memory_space=pltpu.MemorySpace.VMEM),
            pl.BlockSpec(memory_space=pltpu.MemorySpace.VMEM),
            pl.BlockSpec(memory_space=pltpu.MemorySpace.VMEM),
            pl.BlockSpec(memory_space=pltpu.MemorySpace.VMEM),
        ],
        out_specs=[
            pl.BlockSpec(memory_space=pltpu.MemorySpace.HBM),
            pl.BlockSpec(memory_space=pltpu.MemorySpace.VMEM),
            pl.BlockSpec(memory_space=pltpu.MemorySpace.VMEM),
        ],
        out_shape=[
            jax.ShapeDtypeStruct((B, T, H), jnp.float32),
            jax.ShapeDtypeStruct((B, H), jnp.float32),
            jax.ShapeDtypeStruct((B, H), jnp.float32),
        ],
        scratch_shapes=[
            pltpu.VMEM((2, C, B, D), jnp.float32),
            pltpu.VMEM((2, C, B, H), jnp.float32),
            pltpu.VMEM((C * B, G), jnp.bfloat16),
            pltpu.VMEM((C, H), jnp.float32),
            pltpu.VMEM((B, H), jnp.float32),
            pltpu.VMEM((B, H), jnp.float32),
            pltpu.SemaphoreType.DMA((2,)),
            pltpu.SemaphoreType.DMA((2,)),
            pltpu.SemaphoreType.DMA,
        ],
    )(inputs, permutation.astype(jnp.int32), sorted_lengths[:, None],
      W_ih.T, W_hh.T, b_ih[None, :], b_hh[None, :])

    return outputs, hs[None], cs[None], restoration
